# baseline (speedup 1.0000x reference)
import functools

import jax
import jax.numpy as jnp
from jax import lax
from jax.experimental import pallas as pl
from jax.experimental.pallas import tpu as pltpu
from jax.experimental.pallas import tpu_sc as plsc

_B = 16384
_D = 64
_DP = 128
_L = 16
_NC = 2
_NS = 16
_NW = _NC * _NS
_TB = _B // _NW
_C = 128
_NCH = _TB // _C


def _vsqrt(x):
    xi = lax.bitcast_convert_type(x, jnp.int32)
    yi = jnp.int32(0x5F3759DF) - (xi >> 1)
    y = lax.bitcast_convert_type(yi, jnp.float32)
    for _ in range(3):
        y = y * (1.5 - 0.5 * x * y * y)
    return x * y


def _signish(x):
    return jnp.minimum(jnp.maximum(x * 1e12, -1.0), 1.0)


def _sc_partials(entity_emb, relation_emb, idx, par):
    mesh = plsc.VectorSubcoreMesh(core_axis_name="c", subcore_axis_name="s")

    @functools.partial(
        pl.kernel,
        mesh=mesh,
        compiler_params=pltpu.CompilerParams(
            needs_layout_passes=False, use_tc_tiling_on_sc=True
        ),
        out_type=jax.ShapeDtypeStruct((_NW * _L,), jnp.float32),
        scratch_types=[
            pltpu.VMEM((6, _NCH, _C), jnp.int32),
            pltpu.VMEM((6 * _NCH * _C,), jnp.int32),
        ]
        + [pltpu.VMEM((_C, _DP), jnp.float32) for _ in range(6)]
        + [
            pltpu.VMEM((_L,), jnp.float32),
            pltpu.SemaphoreType.DMA,
        ],
    )
    def k(ent_hbm, rel_hbm, idx_hbm, par_hbm, out_hbm, idxv, parv, *rest):
        rows = rest[0:6]
        lossbuf, sem0 = rest[6], rest[7]
        wid = lax.axis_index("s") * _NC + lax.axis_index("c")

        for j in range(6):
            pltpu.sync_copy(idx_hbm.at[j, wid], idxv.at[j])
            pltpu.sync_copy(
                par_hbm.at[j, wid],
                parv.at[pl.ds(j * _NCH * _C, _NCH * _C)],
            )

        def start(u):
            handles = []
            for j in range(6):
                tbl = rel_hbm if j in (1, 4) else ent_hbm
                handles.append(
                    pltpu.async_copy(tbl.at[idxv.at[j, u]], rows[j], sem0)
                )
            return handles

        def compute(u, loss_acc):
            def gbody(g, carry):
                base = g * _L
                pvs = [
                    parv[pl.ds((j * _NCH + u) * _C + base, _L)] * _D
                    for j in range(6)
                ]
                for lane in range(_L):
                    i = base + lane
                    z = jnp.zeros((_L,), jnp.float32)
                    accp, accn = z, z
                    offs = [pv[lane] for pv in pvs]
                    for kk in range(_D // _L):
                        sp = (
                            _signish(rows[0][i, pl.ds(offs[0] + kk * _L, _L)])
                            + rows[1][i, pl.ds(offs[1] + kk * _L, _L)]
                            - _signish(rows[2][i, pl.ds(offs[2] + kk * _L, _L)])
                        )
                        sn = (
                            _signish(rows[3][i, pl.ds(offs[3] + kk * _L, _L)])
                            + rows[4][i, pl.ds(offs[4] + kk * _L, _L)]
                            - _signish(rows[5][i, pl.ds(offs[5] + kk * _L, _L)])
                        )
                        accp = accp + sp * sp
                        accn = accn + sn * sn
                    term = jnp.maximum(
                        _vsqrt(jnp.sum(accp)) - _vsqrt(jnp.sum(accn)) + 1.0, 0.0
                    )
                    carry = carry + term
                return carry

            return lax.fori_loop(0, _C // _L, gbody, loss_acc)

        def ubody(u, loss):
            handles = start(u)
            for h in handles:
                h.wait()
            return compute(u, loss)

        loss = lax.fori_loop(0, _NCH, ubody, jnp.float32(0.0))

        lane = lax.iota(jnp.int32, _L)
        lossbuf[...] = jnp.where(lane == 0, loss, jnp.float32(0.0))
        pltpu.sync_copy(lossbuf, out_hbm.at[pl.ds(wid * _L, _L)])

    return k(entity_emb, relation_emb, idx, par)


def _tc_reduce(partials):
    def body(x_ref, o_ref):
        o_ref[...] = jnp.full((1, 1), jnp.sum(x_ref[...]))

    return pl.pallas_call(
        body,
        out_shape=jax.ShapeDtypeStruct((1, 1), jnp.float32),
    )(partials)


def kernel(pos_exmpls, neg_exmpls, entity_emb, relation_emb):
    pos = pos_exmpls.astype(jnp.int32)
    neg = neg_exmpls.astype(jnp.int32)
    rows_all = jnp.concatenate([pos.T, neg.T], axis=0)
    idx = (rows_all >> 1).reshape(6, _NW, _NCH, _C)
    par = (rows_all & 1).reshape(6, _NW, _NCH * _C)
    ent_p = entity_emb.reshape(500000, _DP)
    rel_p = relation_emb.reshape(500000, _DP)
    partials = _sc_partials(ent_p, rel_p, idx, par)
    return _tc_reduce(partials)[0, 0]


# speedup vs baseline: 1.0823x; 1.0823x over previous
import functools

import jax
import jax.numpy as jnp
from jax import lax
from jax.experimental import pallas as pl
from jax.experimental.pallas import tpu as pltpu
from jax.experimental.pallas import tpu_sc as plsc

_B = 16384
_D = 64
_DP = 128
_L = 16
_NC = 2
_NS = 16
_NW = _NC * _NS
_TB = _B // _NW
_C = 64
_NCH = _TB // _C


def _vsqrt(x):
    xi = lax.bitcast_convert_type(x, jnp.int32)
    yi = jnp.int32(0x5F3759DF) - (xi >> 1)
    y = lax.bitcast_convert_type(yi, jnp.float32)
    for _ in range(3):
        y = y * (1.5 - 0.5 * x * y * y)
    return x * y


def _signish(x):
    return jnp.minimum(jnp.maximum(x * 1e12, -1.0), 1.0)


def _sc_partials(entity_emb, relation_emb, idx):
    mesh = plsc.VectorSubcoreMesh(core_axis_name="c", subcore_axis_name="s")

    @functools.partial(
        pl.kernel,
        mesh=mesh,
        compiler_params=pltpu.CompilerParams(
            needs_layout_passes=False, use_tc_tiling_on_sc=True
        ),
        out_type=jax.ShapeDtypeStruct((_NW * _L,), jnp.float32),
        scratch_types=[pltpu.VMEM((6, _NCH, _C), jnp.int32)]
        + [pltpu.VMEM((_C, _DP), jnp.float32) for _ in range(12)]
        + [
            pltpu.VMEM((_L,), jnp.float32),
            pltpu.SemaphoreType.DMA,
            pltpu.SemaphoreType.DMA,
        ],
    )
    def k(ent_hbm, rel_hbm, idx_hbm, out_hbm, idxv, *rest):
        rows = [rest[0:6], rest[6:12]]
        lossbuf, sem0, sem1 = rest[12], rest[13], rest[14]
        wid = lax.axis_index("s") * _NC + lax.axis_index("c")
        sems = [sem0, sem1]

        for j in range(6):
            pltpu.sync_copy(idx_hbm.at[j, wid], idxv.at[j])

        def start(u):
            p = u % 2
            handles = []
            for j in range(6):
                tbl = rel_hbm if j in (1, 4) else ent_hbm
                handles.append(
                    pltpu.async_copy(tbl.at[idxv.at[j, u]], rows[p][j], sems[p])
                )
            return handles

        def compute(u, loss_acc):
            p = u % 2

            def ibody(i, carry):
                z = jnp.zeros((_L,), jnp.float32)
                accp, accn = z, z
                for kk in range(_D // _L):
                    sl = pl.ds(kk * _L, _L)
                    sp = (
                        _signish(rows[p][0][i, sl])
                        + rows[p][1][i, sl]
                        - _signish(rows[p][2][i, sl])
                    )
                    sn = (
                        _signish(rows[p][3][i, sl])
                        + rows[p][4][i, sl]
                        - _signish(rows[p][5][i, sl])
                    )
                    accp = accp + sp * sp
                    accn = accn + sn * sn
                term = jnp.maximum(
                    _vsqrt(jnp.sum(accp)) - _vsqrt(jnp.sum(accn)) + 1.0, 0.0
                )
                return carry + term

            return lax.fori_loop(0, _C, ibody, loss_acc, unroll=4)

        copies = start(0)
        loss = jnp.float32(0.0)
        for u in range(_NCH):
            for h in copies:
                h.wait()
            copies = start(u + 1) if u + 1 < _NCH else []
            loss = compute(u, loss)

        lane = lax.iota(jnp.int32, _L)
        lossbuf[...] = jnp.where(lane == 0, loss, jnp.float32(0.0))
        pltpu.sync_copy(lossbuf, out_hbm.at[pl.ds(wid * _L, _L)])

    return k(entity_emb, relation_emb, idx)


def _tc_reduce(partials):
    def body(x_ref, o_ref):
        o_ref[...] = jnp.full((1, 1), jnp.sum(x_ref[...]))

    return pl.pallas_call(
        body,
        out_shape=jax.ShapeDtypeStruct((1, 1), jnp.float32),
    )(partials)


def kernel(pos_exmpls, neg_exmpls, entity_emb, relation_emb):
    pos = pos_exmpls.astype(jnp.int32)
    neg = neg_exmpls.astype(jnp.int32)
    idx = jnp.concatenate([pos.T, neg.T], axis=0).reshape(6, _NW, _NCH, _C)
    ent_p = jnp.pad(entity_emb, ((0, 0), (0, _DP - _D)))
    rel_p = jnp.pad(relation_emb, ((0, 0), (0, _DP - _D)))
    partials = _sc_partials(ent_p, rel_p, idx)
    return _tc_reduce(partials)[0, 0]
